# baseline (speedup 1.0000x reference)
import jax
import jax.numpy as jnp
from jax import lax
from jax.experimental import pallas as pl
from jax.experimental.pallas import tpu as pltpu
from jax.experimental.pallas import tpu_sc as plsc

VOCAB = 1000000
EMB = 16
MAX_NORM = 100.0

NC, NS, L = 2, 16, 16
NW = NC * NS

SEQ = 50
NB = 16384
G = 128
TCW = NB // G // NW
CHUNK = TCW * G

AU = 512
NFULL = (VOCAB // G) // (AU // G)
UW = 61
TAIL0 = NFULL * AU


def _rsqrt(x):
    i = plsc.bitcast(x, jnp.int32)
    i = jnp.int32(0x5F3759DF) - lax.shift_right_arithmetic(i, 1)
    y = plsc.bitcast(i, jnp.float32)
    for _ in range(2):
        y = y * (1.5 - 0.5 * x * y * y)
    return y


ANB = 2


def _body_detile(wt_hbm, wtail_hbm, wlin_hbm, *scr):
    bins = scr[0:ANB]
    bouts = scr[ANB:2 * ANB]
    gsems = scr[2 * ANB:3 * ANB]
    wsems = scr[3 * ANB:4 * ANB]
    wid = lax.axis_index("c") * NS + lax.axis_index("s")
    base_u = wid * UW

    riota = lax.iota(jnp.int32, L)

    def fire_in(gu, b):
        pltpu.async_copy(
            wt_hbm.at[:, :, pl.ds(gu * AU, AU)], bins[b], gsems[b])

    def drain_in(b):
        pltpu.make_async_copy(
            wt_hbm.at[:, :, pl.ds(0, AU)], bins[b], gsems[b]).wait()

    def fire_write(gu, b):
        pltpu.async_copy(
            bouts[b], wlin_hbm.at[pl.ds(gu * (AU * EMB // 128), 64)],
            wsems[b])

    def drain_write(b):
        pltpu.make_async_copy(
            bouts[b], wlin_hbm.at[pl.ds(0, 64)], wsems[b]).wait()

    def transpose(b):
        bin_v = bins[b]
        bout_v = bouts[b]

        @plsc.parallel_loop(0, AU // L, unroll=4)
        def subgroup(sg):
            row = sg * L + riota
            q = lax.shift_right_logical(row, 3)
            rm = lax.shift_left(jnp.bitwise_and(row, 7), 4)
            for f in range(EMB):
                cf = bin_v[f // 8, f % 8, pl.ds(sg * L, L)]
                plsc.store_scatter(bout_v, [q, rm + f], cf)

    for k in range(ANB - 1):
        fire_in(base_u + k, k)

    @pl.loop(0, UW - 1, step=ANB)
    def unit_ring(u0):
        for db in range(ANB):
            u = u0 + db
            b = db
            drain_in(b)

            @pl.when(u + ANB - 1 < UW)
            def _():
                fire_in(base_u + u + ANB - 1, (b + ANB - 1) % ANB)

            @pl.when(u >= ANB)
            def _():
                drain_write(b)

            transpose(b)
            fire_write(base_u + u, b)

    drain_write(0)
    drain_in(0)
    transpose(0)
    fire_write(base_u + UW - 1, 0)
    for b in range(ANB):
        drain_write(b)

    @pl.when(wid == NW - 1)
    def _():
        fire_in(NFULL - 1, 0)
        drain_in(0)
        transpose(0)
        fire_write(NFULL - 1, 0)
        drain_write(0)

        pltpu.sync_copy(wtail_hbm, bouts[0].at[pl.ds(0, 8)])
        pltpu.sync_copy(
            bouts[0].at[pl.ds(0, 8)],
            wlin_hbm.at[pl.ds(TAIL0 * EMB // 128, 8)])


BNB = 2


def _body_lookup(table_hbm, idx_hbm, out_hbm, idx_v, *scr):
    rows = scr[0:BNB]
    outt = scr[BNB:2 * BNB]
    gsems = scr[2 * BNB:3 * BNB]
    wsems = scr[3 * BNB:4 * BNB]
    wid = lax.axis_index("c") * NS + lax.axis_index("s")

    pltpu.sync_copy(idx_hbm.at[:, pl.ds(wid * TCW, TCW)], idx_v)

    riota = lax.iota(jnp.int32, L)

    def fire_gather(s, b):
        for j in range(TCW):
            pltpu.async_copy(
                table_hbm.at[idx_v.at[s, j]],
                rows[b].at[pl.ds(j * G, G)], gsems[b])

    def drain_gather(b):
        for j in range(TCW):
            pltpu.make_async_copy(
                table_hbm.at[pl.ds(0, G)],
                rows[b].at[pl.ds(j * G, G)], gsems[b]).wait()

    def fire_write(s, b):
        for tr in range(2):
            pltpu.async_copy(
                outt[b].at[tr],
                out_hbm.at[s, pl.ds(tr * 128 + wid * TCW, TCW)], wsems[b])

    def drain_write(b):
        for tr in range(2):
            pltpu.make_async_copy(
                outt[b].at[tr],
                out_hbm.at[0, pl.ds(0, TCW)], wsems[b]).wait()

    def compute(b):
        rows_v = rows[b]
        outt_v = outt[b]

        @plsc.parallel_loop(0, CHUNK // L, unroll=2)
        def subgroup(sg):
            row_idx = sg * L + riota
            j = sg // (G // L)
            col = (sg % (G // L)) * L
            cols = []
            ss = jnp.zeros((L,), jnp.float32)
            for f in range(EMB):
                cf = plsc.load_gather(
                    rows_v, [row_idx, jnp.full((L,), f, jnp.int32)])
                cols.append(cf)
                ss = ss + cf * cf
            ssc = jnp.maximum(ss, 1e-14)
            scale = jnp.minimum(1.0, MAX_NORM * _rsqrt(ssc))
            for f in range(EMB):
                outt_v[f // 8, j, pl.ds((f % 8) * 128 + col, L)] = (
                    cols[f] * scale)

    for k in range(BNB - 1):
        fire_gather(k, k)

    @pl.loop(0, SEQ, step=BNB)
    def unit_ring(s0):
        for db in range(BNB):
            s = s0 + db
            b = db
            drain_gather(b)

            @pl.when(s + BNB - 1 < SEQ)
            def _():
                fire_gather(s + BNB - 1, (b + BNB - 1) % BNB)

            @pl.when(s >= BNB)
            def _():
                drain_write(b)

            compute(b)
            fire_write(s, b)

    for b in range(BNB):
        drain_write(b)


@jax.jit
def _run(x, W):
    wt3 = W.T.reshape(2, 8, VOCAB)
    wtail = lax.slice(W, (TAIL0, 0), (VOCAB, EMB)).reshape(8, 128)
    xt3 = x.T.astype(jnp.int32).reshape(SEQ, NB // G, G)

    detile = pl.kernel(
        _body_detile,
        out_type=jax.ShapeDtypeStruct((VOCAB * EMB // 128, 128),
                                      jnp.float32),
        mesh=plsc.VectorSubcoreMesh(core_axis_name="c", subcore_axis_name="s"),
        scratch_types=(
            [pltpu.VMEM((2, 8, AU), jnp.float32)] * ANB
            + [pltpu.VMEM((64, 128), jnp.float32)] * ANB
            + [pltpu.SemaphoreType.DMA] * (2 * ANB)
        ),
        compiler_params=pltpu.CompilerParams(
            needs_layout_passes=False, use_tc_tiling_on_sc=True),
    )
    w_rm = detile(wt3, wtail).reshape(VOCAB, EMB)

    lookup = pl.kernel(
        _body_lookup,
        out_type=jax.ShapeDtypeStruct((SEQ, 256, 1024), jnp.float32),
        mesh=plsc.VectorSubcoreMesh(core_axis_name="c", subcore_axis_name="s"),
        scratch_types=(
            [pltpu.VMEM((SEQ, TCW, G), jnp.int32)]
            + [pltpu.VMEM((CHUNK, EMB), jnp.float32)] * BNB
            + [pltpu.VMEM((2, TCW, 1024), jnp.float32)] * BNB
            + [pltpu.SemaphoreType.DMA] * (2 * BNB)
        ),
        compiler_params=pltpu.CompilerParams(
            needs_layout_passes=False, use_tc_tiling_on_sc=False),
    )
    out = lookup(w_rm, xt3)

    out = out.reshape(SEQ, 2, 128, 8, 128)
    out = out.transpose(2, 4, 0, 1, 3)
    return out.reshape(NB, SEQ, EMB)


def kernel(x, W):
    return _run(x, W)


# speedup vs baseline: 1.0825x; 1.0825x over previous
import jax
import jax.numpy as jnp
from jax import lax
from jax.experimental import pallas as pl
from jax.experimental.pallas import tpu as pltpu
from jax.experimental.pallas import tpu_sc as plsc

VOCAB = 1000000
EMB = 16
MAX_NORM = 100.0

NC, NS, L = 2, 16, 16
NW = NC * NS

SEQ = 50
NB = 16384
G = 128
TCW = NB // G // NW
CHUNK = TCW * G

AU = 512
NFULL = (VOCAB // G) // (AU // G)
UW = 61
TAIL0 = NFULL * AU


def _rsqrt(x):
    i = plsc.bitcast(x, jnp.int32)
    i = jnp.int32(0x5F3759DF) - lax.shift_right_arithmetic(i, 1)
    y = plsc.bitcast(i, jnp.float32)
    for _ in range(2):
        y = y * (1.5 - 0.5 * x * y * y)
    return y


ANB = 6


def _body_detile(wt_hbm, wtail_hbm, wlin_hbm, *scr):
    bins = scr[0:ANB]
    bouts = scr[ANB:2 * ANB]
    gsems = scr[2 * ANB:3 * ANB]
    wsems = scr[3 * ANB:4 * ANB]
    wid = lax.axis_index("c") * NS + lax.axis_index("s")
    base_u = wid * UW

    riota = lax.iota(jnp.int32, L)

    def fire_in(gu, b):
        pltpu.async_copy(
            wt_hbm.at[:, :, pl.ds(gu * AU, AU)], bins[b], gsems[b])

    def drain_in(b):
        pltpu.make_async_copy(
            wt_hbm.at[:, :, pl.ds(0, AU)], bins[b], gsems[b]).wait()

    def fire_write(gu, b):
        pltpu.async_copy(
            bouts[b], wlin_hbm.at[pl.ds(gu * (AU * EMB // 128), 64)],
            wsems[b])

    def drain_write(b):
        pltpu.make_async_copy(
            bouts[b], wlin_hbm.at[pl.ds(0, 64)], wsems[b]).wait()

    def transpose(b):
        bin_v = bins[b]
        bout_v = bouts[b]

        @plsc.parallel_loop(0, AU // L, unroll=4)
        def subgroup(sg):
            row = sg * L + riota
            q = lax.shift_right_logical(row, 3)
            rm = lax.shift_left(jnp.bitwise_and(row, 7), 4)
            for f in range(EMB):
                cf = bin_v[f // 8, f % 8, pl.ds(sg * L, L)]
                plsc.store_scatter(bout_v, [q, rm + f], cf)

    for k in range(ANB - 1):
        fire_in(base_u + k, k)

    @pl.loop(0, UW - 1, step=ANB)
    def unit_ring(u0):
        for db in range(ANB):
            u = u0 + db
            b = db
            drain_in(b)

            @pl.when(u + ANB - 1 < UW)
            def _():
                fire_in(base_u + u + ANB - 1, (b + ANB - 1) % ANB)

            @pl.when(u >= ANB)
            def _():
                drain_write(b)

            transpose(b)
            fire_write(base_u + u, b)

    drain_write(0)
    drain_in(0)
    transpose(0)
    fire_write(base_u + UW - 1, 0)
    for b in range(ANB):
        drain_write(b)

    @pl.when(wid == NW - 1)
    def _():
        fire_in(NFULL - 1, 0)
        drain_in(0)
        transpose(0)
        fire_write(NFULL - 1, 0)
        drain_write(0)

        pltpu.sync_copy(wtail_hbm, bouts[0].at[pl.ds(0, 8)])
        pltpu.sync_copy(
            bouts[0].at[pl.ds(0, 8)],
            wlin_hbm.at[pl.ds(TAIL0 * EMB // 128, 8)])


BNB = 2


def _body_lookup(table_hbm, idx_hbm, out_hbm, idx_v, *scr):
    rows = scr[0:BNB]
    outt = scr[BNB:2 * BNB]
    gsems = scr[2 * BNB:3 * BNB]
    wsems = scr[3 * BNB:4 * BNB]
    wid = lax.axis_index("c") * NS + lax.axis_index("s")

    pltpu.sync_copy(idx_hbm.at[:, pl.ds(wid * TCW, TCW)], idx_v)

    riota = lax.iota(jnp.int32, L)

    def fire_gather(s, b):
        for j in range(TCW):
            pltpu.async_copy(
                table_hbm.at[idx_v.at[s, j]],
                rows[b].at[pl.ds(j * G, G)], gsems[b])

    def drain_gather(b):
        for j in range(TCW):
            pltpu.make_async_copy(
                table_hbm.at[pl.ds(0, G)],
                rows[b].at[pl.ds(j * G, G)], gsems[b]).wait()

    def fire_write(s, b):
        for tr in range(2):
            pltpu.async_copy(
                outt[b].at[tr],
                out_hbm.at[s, pl.ds(tr * 128 + wid * TCW, TCW)], wsems[b])

    def drain_write(b):
        for tr in range(2):
            pltpu.make_async_copy(
                outt[b].at[tr],
                out_hbm.at[0, pl.ds(0, TCW)], wsems[b]).wait()

    def compute(b):
        rows_v = rows[b]
        outt_v = outt[b]

        @plsc.parallel_loop(0, CHUNK // L, unroll=2)
        def subgroup(sg):
            row_idx = sg * L + riota
            j = sg // (G // L)
            col = (sg % (G // L)) * L
            cols = []
            ss = jnp.zeros((L,), jnp.float32)
            for f in range(EMB):
                cf = plsc.load_gather(
                    rows_v, [row_idx, jnp.full((L,), f, jnp.int32)])
                cols.append(cf)
                ss = ss + cf * cf
            ssc = jnp.maximum(ss, 1e-14)
            scale = jnp.minimum(1.0, MAX_NORM * _rsqrt(ssc))
            for f in range(EMB):
                outt_v[f // 8, j, pl.ds((f % 8) * 128 + col, L)] = (
                    cols[f] * scale)

    for k in range(BNB - 1):
        fire_gather(k, k)

    @pl.loop(0, SEQ, step=BNB)
    def unit_ring(s0):
        for db in range(BNB):
            s = s0 + db
            b = db
            drain_gather(b)

            @pl.when(s + BNB - 1 < SEQ)
            def _():
                fire_gather(s + BNB - 1, (b + BNB - 1) % BNB)

            @pl.when(s >= BNB)
            def _():
                drain_write(b)

            compute(b)
            fire_write(s, b)

    for b in range(BNB):
        drain_write(b)


@jax.jit
def _run(x, W):
    wt3 = W.T.reshape(2, 8, VOCAB)
    wtail = lax.slice(W, (TAIL0, 0), (VOCAB, EMB)).reshape(8, 128)
    xt3 = x.T.astype(jnp.int32).reshape(SEQ, NB // G, G)

    detile = pl.kernel(
        _body_detile,
        out_type=jax.ShapeDtypeStruct((VOCAB * EMB // 128, 128),
                                      jnp.float32),
        mesh=plsc.VectorSubcoreMesh(core_axis_name="c", subcore_axis_name="s"),
        scratch_types=(
            [pltpu.VMEM((2, 8, AU), jnp.float32)] * ANB
            + [pltpu.VMEM((64, 128), jnp.float32)] * ANB
            + [pltpu.SemaphoreType.DMA] * (2 * ANB)
        ),
        compiler_params=pltpu.CompilerParams(
            needs_layout_passes=False, use_tc_tiling_on_sc=True),
    )
    w_rm = detile(wt3, wtail).reshape(VOCAB, EMB)

    lookup = pl.kernel(
        _body_lookup,
        out_type=jax.ShapeDtypeStruct((SEQ, 256, 1024), jnp.float32),
        mesh=plsc.VectorSubcoreMesh(core_axis_name="c", subcore_axis_name="s"),
        scratch_types=(
            [pltpu.VMEM((SEQ, TCW, G), jnp.int32)]
            + [pltpu.VMEM((CHUNK, EMB), jnp.float32)] * BNB
            + [pltpu.VMEM((2, TCW, 1024), jnp.float32)] * BNB
            + [pltpu.SemaphoreType.DMA] * (2 * BNB)
        ),
        compiler_params=pltpu.CompilerParams(
            needs_layout_passes=False, use_tc_tiling_on_sc=False),
    )
    out = lookup(w_rm, xt3)

    out = out.reshape(SEQ, 2, 128, 8, 128)
    out = out.transpose(2, 4, 0, 1, 3)
    return out.reshape(NB, SEQ, EMB)


def kernel(x, W):
    return _run(x, W)
